# Initial kernel scaffold; baseline (speedup 1.0000x reference)
import jax
import jax.numpy as jnp
from jax.experimental import pallas as pl


def kernel(origins, directions, sdf_values):
    raise NotImplementedError("write your pallas kernel here")


# speedup vs baseline: 1.7844x; 1.7844x over previous
import functools

import jax
import jax.numpy as jnp
import numpy as np
from jax import lax
from jax.experimental import pallas as pl
from jax.experimental.pallas import tpu as pltpu
from jax.experimental.pallas import tpu_sc as plsc

_N = 256
_B = 16384
_STEPS = 8

_MIN_B = jnp.array([-1.0, -1.0, -1.0], jnp.float32)
_MAX_B = jnp.array([1.0, 1.0, 1.0], jnp.float32)
_VOXEL = (_MAX_B - _MIN_B) / jnp.array([_N - 1.0] * 3, jnp.float32)
_UPPER = jnp.array([_N - 1] * 3, jnp.int32)
_OFFSETS = jnp.array([[0, 0, 0], [1, 0, 0], [0, 1, 0], [1, 1, 0],
                      [0, 0, 1], [1, 0, 1], [0, 1, 1], [1, 1, 1]], jnp.int32)

_NW = 32
_GPW = 64 * _B // _NW
_ROWS = _GPW // 128


def _trajectory(origins, directions):
    inv_dir = 1.0 / directions
    t1 = (_MIN_B - origins) * inv_dir
    t2 = (_MAX_B - origins) * inv_dir
    t_near = jnp.max(jnp.minimum(t1, t2), axis=-1)
    t_far = jnp.min(jnp.maximum(t1, t2), axis=-1)
    t_near = t_near * (t_near > 0)
    valid = t_near <= t_far
    t_near = jnp.where(valid, t_near, 0.0)
    t_far = jnp.where(valid, t_far, -1.0)
    current_pos = origins + t_near[:, None] * directions
    idx = jnp.floor((current_pos - _MIN_B) / _VOXEL).astype(jnp.int32)
    step = jnp.sign(directions).astype(jnp.int32)
    t_delta = _VOXEL / jnp.abs(directions)
    next_boundary = (idx + jnp.maximum(step, 0)).astype(jnp.float32) * _VOXEL + _MIN_B
    t_next = (next_boundary - origins) / directions
    q_l, act_l = [], []
    for _ in range(_STEPS):
        inside = jnp.all((idx >= 0) & (idx < _UPPER[None, :]), axis=-1)
        qidx = jnp.floor((current_pos - _MIN_B) / _VOXEL).astype(jnp.int32)
        q_l.append(qidx)
        act_l.append(valid & inside)
        t_min = jnp.min(t_next, axis=-1, keepdims=True)
        mask = (t_next < t_far[:, None]) & (t_next == t_min)
        idx = idx + step * mask.astype(jnp.int32)
        current_pos = origins + t_min * directions
        t_next = t_next + t_delta * mask.astype(jnp.float32)
    q = jnp.stack(q_l)
    a = jnp.stack(act_l)
    q, a = lax.optimization_barrier((q, a))
    corners = q[:, :, None, :] + _OFFSETS[None, None, :, :]
    cc = jnp.clip(corners, 0, _UPPER[None, None, None, :])
    flat = cc[..., 0] * (_N * _N) + cc[..., 1] * _N + cc[..., 2]
    gidx = flat.reshape(_NW, _ROWS, 128)
    return gidx, q, a


def _sc_gather_kernel(table_hbm, idx_hbm, out_hbm, idx_v, rows_v, sem):
    wid = lax.axis_index("s") * 2 + lax.axis_index("c")
    pltpu.sync_copy(idx_hbm.at[wid], idx_v)
    nbatch = 16
    rpb = _ROWS // nbatch

    def _fire(j):
        for u in range(rpb):
            r = j * rpb + u
            pltpu.async_copy(table_hbm.at[idx_v.at[r]], rows_v.at[r], sem)

    def _drain(j):
        for u in range(rpb):
            r = j * rpb + u
            pltpu.make_async_copy(table_hbm.at[idx_v.at[r]], rows_v.at[r],
                                  sem).wait()

    _fire(0)

    def _body(j, carry):
        _fire(j)
        _drain(j - 1)
        return carry

    lax.fori_loop(1, nbatch, _body, 0)
    _drain(nbatch - 1)
    pltpu.sync_copy(rows_v, out_hbm.at[wid])


def _sc_gather(flat_sdf, idx):
    gk = functools.partial(
        pl.kernel,
        mesh=plsc.VectorSubcoreMesh(core_axis_name="c", subcore_axis_name="s"),
        out_type=jax.ShapeDtypeStruct((_NW, _ROWS, 128), jnp.float32),
        scratch_types=[
            pltpu.VMEM((_ROWS, 128), jnp.int32),
            pltpu.VMEM((_ROWS, 128), jnp.float32),
            pltpu.SemaphoreType.DMA,
        ],
    )(_sc_gather_kernel)
    return gk(flat_sdf, idx)


def _solve(origins, directions, gathered, q, acts):
    gathered, q, acts = lax.optimization_barrier((gathered, q, acts))
    sdf3 = gathered.reshape(_STEPS, _B, 8)
    hit = jnp.zeros((_B,), bool)
    intersection = jnp.zeros_like(origins)
    normal = jnp.zeros_like(origins)
    for s in range(_STEPS):
        sdf = sdf3[s]
        corners = q[s][:, None, :] + _OFFSETS[None, :, :]
        positions = _MIN_B + corners.astype(jnp.float32) * _VOXEL
        act = acts[s] & (~hit)
        s000, s100, s010, s110, s001, s101, s011, s111 = [sdf[:, i] for i in range(8)]
        px000 = positions[:, 0, 0]; py000 = positions[:, 0, 1]; pz000 = positions[:, 0, 2]
        px100 = positions[:, 1, 0]; py010 = positions[:, 2, 1]; pz001 = positions[:, 4, 2]
        ox = (origins[:, 0] - px000) / (px100 - px000)
        oy = (origins[:, 1] - py000) / (py010 - py000)
        oz = (origins[:, 2] - pz000) / (pz001 - pz000)
        dx = directions[:, 0] / (px100 - px000)
        dy = directions[:, 1] / (py010 - py000)
        dz = directions[:, 2] / (pz001 - pz000)
        k0 = s000; k1 = s100 - s000; k2 = s010 - s000; k3 = s110 - s010 - k1
        k4 = k0 - s001
        a = s101 - s001
        k5 = k1 - a
        k6 = k2 - (s011 - s001)
        k7 = k3 - (s111 - s011 - a)
        m0 = ox * oy; m1 = dx * dy; m2 = ox * dy + oy * dx
        m3 = k5 * oz - k1; m4 = k6 * oz - k2; m5 = k7 * oz - k3
        c0 = k4 * oz - k0 + ox * m3 + oy * m4 + m0 * m5
        c1 = dx * m3 + dy * m4 + m2 * m5 + dz * (k4 + k5 * ox + k6 * oy + k7 * m0)
        c2 = m1 * m5 + dz * (k5 * dx + k6 * dy + k7 * m2)
        c3 = k7 * m1 * dz

        lin_valid = c1 > 1e-19
        c1s = jnp.where(lin_valid, c1, 1.0)
        t_lin = -c0 / c1s
        use_lin = c2 <= 1e-4
        c2s = jnp.where(use_lin, 1.0, c2)
        disc = c1 ** 2 - 4.0 * c2 * c0
        sq = jnp.sqrt(jnp.maximum(disc, 0.0))
        r1 = (-c1 + sq) / (2.0 * c2s)
        r2 = (-c1 - sq) / (2.0 * c2s)
        rmin = jnp.minimum(r1, r2)
        t_q = jnp.where(use_lin, t_lin, rmin)
        v_q = jnp.where(use_lin, lin_valid, disc > 0)

        use_quad = c3 <= 1e-4
        c3s = jnp.where(use_quad, 1.0, c3)
        aa = c2 / c3s
        bb = c1 / c3s
        cc_ = c0 / c3s
        Q = (aa ** 2 - 3.0 * bb) / 9.0
        Rr = (2.0 * aa ** 3 - 9.0 * aa * bb + 27.0 * cc_) / 54.0
        trig = Rr ** 2 < Q ** 3
        Qp = jnp.maximum(Q, 1e-12)
        ratio = jnp.clip(Rr / jnp.sqrt(Qp ** 3), -0.999999, 0.999999)
        theta = jnp.arccos(ratio)
        sqrt_Q = jnp.sqrt(Qp)
        t_trig = -2.0 * sqrt_Q * jnp.cos(theta / 3.0) - aa / 3.0
        arg = jnp.maximum(Rr ** 2 - Q ** 3, 0.0)
        base = jnp.maximum(jnp.abs(Rr) + jnp.sqrt(arg), 1e-12)
        A = -jnp.sign(Rr) * base ** (1.0 / 3.0)
        nonzero_A = jnp.abs(A) > 0
        As = jnp.where(nonzero_A, A, 1.0)
        Bv = jnp.where(nonzero_A, Q / As, 0.0)
        t_card = A + Bv - aa / 3.0
        t_cub = jnp.where(trig, t_trig, t_card)
        t = jnp.where(use_quad, t_q, t_cub)
        root_valid = jnp.where(use_quad, v_q, jnp.ones_like(v_q))

        new_hit = act & root_valid
        inter = origins + t[:, None] * directions

        x = (inter[:, 0] - px000) / (px100 - px000)
        y = (inter[:, 1] - py000) / (py010 - py000)
        z = (inter[:, 2] - pz000) / (pz001 - pz000)
        y0 = _lerp(y, sdf[:, 1] - sdf[:, 0], sdf[:, 3] - sdf[:, 2])
        y1 = _lerp(y, sdf[:, 5] - sdf[:, 4], sdf[:, 7] - sdf[:, 6])
        df_dx = _lerp(z, y0, y1)
        x0 = _lerp(x, sdf[:, 2] - sdf[:, 0], sdf[:, 3] - sdf[:, 1])
        x1 = _lerp(x, sdf[:, 6] - sdf[:, 4], sdf[:, 7] - sdf[:, 5])
        df_dy = _lerp(z, x0, x1)
        x0 = _lerp(x, sdf[:, 4] - sdf[:, 0], sdf[:, 5] - sdf[:, 1])
        x1 = _lerp(x, sdf[:, 6] - sdf[:, 2], sdf[:, 7] - sdf[:, 3])
        df_dz = _lerp(y, x0, x1)
        nrm = jnp.stack([df_dx, df_dy, df_dz], axis=-1)

        intersection = jnp.where(new_hit[:, None], inter, intersection)
        normal = jnp.where(new_hit[:, None], nrm, normal)
        hit = hit | new_hit
    out = jnp.concatenate([intersection, normal], axis=-1) * hit[:, None].astype(jnp.float32)
    return out


def _lerp(u, a, b):
    return a + u * (b - a)


def kernel(origins, directions, sdf_values):
    gidx, q, acts = _trajectory(origins, directions)
    gathered = _sc_gather(sdf_values.reshape(_N * _N * _N), gidx)
    return _solve(origins, directions, gathered, q, acts)


# speedup vs baseline: 4.3007x; 2.4101x over previous
import functools

import jax
import jax.numpy as jnp
import numpy as np
from jax import lax
from jax.experimental import pallas as pl
from jax.experimental.pallas import tpu as pltpu
from jax.experimental.pallas import tpu_sc as plsc

_N = 256
_B = 16384
_STEPS = 8

_MIN_B = jnp.array([-1.0, -1.0, -1.0], jnp.float32)
_MAX_B = jnp.array([1.0, 1.0, 1.0], jnp.float32)
_VOXEL = (_MAX_B - _MIN_B) / jnp.array([_N - 1.0] * 3, jnp.float32)
_UPPER = jnp.array([_N - 1] * 3, jnp.int32)
_OFFSETS = jnp.array([[0, 0, 0], [1, 0, 0], [0, 1, 0], [1, 1, 0],
                      [0, 0, 1], [1, 0, 1], [0, 1, 1], [1, 1, 1]], jnp.int32)

_NW = 32
_GPW = 64 * _B // _NW
_ROWS = _GPW // 128


def _trajectory(origins, directions):
    inv_dir = 1.0 / directions
    t1 = (_MIN_B - origins) * inv_dir
    t2 = (_MAX_B - origins) * inv_dir
    t_near = jnp.max(jnp.minimum(t1, t2), axis=-1)
    t_far = jnp.min(jnp.maximum(t1, t2), axis=-1)
    t_near = t_near * (t_near > 0)
    valid = t_near <= t_far
    t_near = jnp.where(valid, t_near, 0.0)
    t_far = jnp.where(valid, t_far, -1.0)
    current_pos = origins + t_near[:, None] * directions
    idx = jnp.floor((current_pos - _MIN_B) / _VOXEL).astype(jnp.int32)
    step = jnp.sign(directions).astype(jnp.int32)
    t_delta = _VOXEL / jnp.abs(directions)
    next_boundary = (idx + jnp.maximum(step, 0)).astype(jnp.float32) * _VOXEL + _MIN_B
    t_next = (next_boundary - origins) / directions
    q_l, act_l = [], []
    for _ in range(_STEPS):
        inside = jnp.all((idx >= 0) & (idx < _UPPER[None, :]), axis=-1)
        qidx = jnp.floor((current_pos - _MIN_B) / _VOXEL).astype(jnp.int32)
        q_l.append(qidx)
        act_l.append(valid & inside)
        t_min = jnp.min(t_next, axis=-1, keepdims=True)
        mask = (t_next < t_far[:, None]) & (t_next == t_min)
        idx = idx + step * mask.astype(jnp.int32)
        current_pos = origins + t_min * directions
        t_next = t_next + t_delta * mask.astype(jnp.float32)
    q = jnp.stack(q_l)
    a = jnp.stack(act_l)
    q, a = lax.optimization_barrier((q, a))
    corners = q[:, None, :, :] + _OFFSETS[None, :, None, :]
    cc = jnp.clip(corners, 0, _UPPER[None, None, None, :])
    flat = cc[..., 0] * (_N * _N) + cc[..., 1] * _N + cc[..., 2]
    gidx = flat.reshape(_NW, _ROWS, 128)
    return gidx, q, a


def _sc_gather_kernel(table_hbm, idx_hbm, out_hbm, idx_v, rows_v, sem):
    wid = lax.axis_index("s") * 2 + lax.axis_index("c")
    pltpu.sync_copy(idx_hbm.at[wid], idx_v)
    nbatch = 16
    rpb = _ROWS // nbatch

    def _fire(j):
        for u in range(rpb):
            r = j * rpb + u
            pltpu.async_copy(table_hbm.at[idx_v.at[r]], rows_v.at[r], sem)

    def _drain(j):
        for u in range(rpb):
            r = j * rpb + u
            pltpu.make_async_copy(table_hbm.at[idx_v.at[r]], rows_v.at[r],
                                  sem).wait()

    _fire(0)

    def _body(j, carry):
        _fire(j)
        _drain(j - 1)
        return carry

    lax.fori_loop(1, nbatch, _body, 0)
    _drain(nbatch - 1)
    pltpu.sync_copy(rows_v, out_hbm.at[wid])


def _sc_gather(flat_sdf, idx):
    gk = functools.partial(
        pl.kernel,
        mesh=plsc.VectorSubcoreMesh(core_axis_name="c", subcore_axis_name="s"),
        out_type=jax.ShapeDtypeStruct((_NW, _ROWS, 128), jnp.float32),
        scratch_types=[
            pltpu.VMEM((_ROWS, 128), jnp.int32),
            pltpu.VMEM((_ROWS, 128), jnp.float32),
            pltpu.SemaphoreType.DMA,
        ],
    )(_sc_gather_kernel)
    return gk(flat_sdf, idx)


def _solve(origins, directions, gathered, q, acts):
    gathered, q, acts = lax.optimization_barrier((gathered, q, acts))
    sdf3 = gathered.reshape(_STEPS, 8, _B)
    hit = jnp.zeros((_B,), bool)
    intersection = jnp.zeros_like(origins)
    normal = jnp.zeros_like(origins)
    for s in range(_STEPS):
        sdf = sdf3[s]
        corners = q[s][:, None, :] + _OFFSETS[None, :, :]
        positions = _MIN_B + corners.astype(jnp.float32) * _VOXEL
        act = acts[s] & (~hit)
        s000, s100, s010, s110, s001, s101, s011, s111 = [sdf[i] for i in range(8)]
        px000 = positions[:, 0, 0]; py000 = positions[:, 0, 1]; pz000 = positions[:, 0, 2]
        px100 = positions[:, 1, 0]; py010 = positions[:, 2, 1]; pz001 = positions[:, 4, 2]
        ox = (origins[:, 0] - px000) / (px100 - px000)
        oy = (origins[:, 1] - py000) / (py010 - py000)
        oz = (origins[:, 2] - pz000) / (pz001 - pz000)
        dx = directions[:, 0] / (px100 - px000)
        dy = directions[:, 1] / (py010 - py000)
        dz = directions[:, 2] / (pz001 - pz000)
        k0 = s000; k1 = s100 - s000; k2 = s010 - s000; k3 = s110 - s010 - k1
        k4 = k0 - s001
        a = s101 - s001
        k5 = k1 - a
        k6 = k2 - (s011 - s001)
        k7 = k3 - (s111 - s011 - a)
        m0 = ox * oy; m1 = dx * dy; m2 = ox * dy + oy * dx
        m3 = k5 * oz - k1; m4 = k6 * oz - k2; m5 = k7 * oz - k3
        c0 = k4 * oz - k0 + ox * m3 + oy * m4 + m0 * m5
        c1 = dx * m3 + dy * m4 + m2 * m5 + dz * (k4 + k5 * ox + k6 * oy + k7 * m0)
        c2 = m1 * m5 + dz * (k5 * dx + k6 * dy + k7 * m2)
        c3 = k7 * m1 * dz

        lin_valid = c1 > 1e-19
        c1s = jnp.where(lin_valid, c1, 1.0)
        t_lin = -c0 / c1s
        use_lin = c2 <= 1e-4
        c2s = jnp.where(use_lin, 1.0, c2)
        disc = c1 ** 2 - 4.0 * c2 * c0
        sq = jnp.sqrt(jnp.maximum(disc, 0.0))
        r1 = (-c1 + sq) / (2.0 * c2s)
        r2 = (-c1 - sq) / (2.0 * c2s)
        rmin = jnp.minimum(r1, r2)
        t_q = jnp.where(use_lin, t_lin, rmin)
        v_q = jnp.where(use_lin, lin_valid, disc > 0)

        use_quad = c3 <= 1e-4
        c3s = jnp.where(use_quad, 1.0, c3)
        aa = c2 / c3s
        bb = c1 / c3s
        cc_ = c0 / c3s
        Q = (aa ** 2 - 3.0 * bb) / 9.0
        Rr = (2.0 * aa ** 3 - 9.0 * aa * bb + 27.0 * cc_) / 54.0
        trig = Rr ** 2 < Q ** 3
        Qp = jnp.maximum(Q, 1e-12)
        ratio = jnp.clip(Rr / jnp.sqrt(Qp ** 3), -0.999999, 0.999999)
        theta = jnp.arccos(ratio)
        sqrt_Q = jnp.sqrt(Qp)
        t_trig = -2.0 * sqrt_Q * jnp.cos(theta / 3.0) - aa / 3.0
        arg = jnp.maximum(Rr ** 2 - Q ** 3, 0.0)
        base = jnp.maximum(jnp.abs(Rr) + jnp.sqrt(arg), 1e-12)
        A = -jnp.sign(Rr) * base ** (1.0 / 3.0)
        nonzero_A = jnp.abs(A) > 0
        As = jnp.where(nonzero_A, A, 1.0)
        Bv = jnp.where(nonzero_A, Q / As, 0.0)
        t_card = A + Bv - aa / 3.0
        t_cub = jnp.where(trig, t_trig, t_card)
        t = jnp.where(use_quad, t_q, t_cub)
        root_valid = jnp.where(use_quad, v_q, jnp.ones_like(v_q))

        new_hit = act & root_valid
        inter = origins + t[:, None] * directions

        x = (inter[:, 0] - px000) / (px100 - px000)
        y = (inter[:, 1] - py000) / (py010 - py000)
        z = (inter[:, 2] - pz000) / (pz001 - pz000)
        y0 = _lerp(y, sdf[1] - sdf[0], sdf[3] - sdf[2])
        y1 = _lerp(y, sdf[5] - sdf[4], sdf[7] - sdf[6])
        df_dx = _lerp(z, y0, y1)
        x0 = _lerp(x, sdf[2] - sdf[0], sdf[3] - sdf[1])
        x1 = _lerp(x, sdf[6] - sdf[4], sdf[7] - sdf[5])
        df_dy = _lerp(z, x0, x1)
        x0 = _lerp(x, sdf[4] - sdf[0], sdf[5] - sdf[1])
        x1 = _lerp(x, sdf[6] - sdf[2], sdf[7] - sdf[3])
        df_dz = _lerp(y, x0, x1)
        nrm = jnp.stack([df_dx, df_dy, df_dz], axis=-1)

        intersection = jnp.where(new_hit[:, None], inter, intersection)
        normal = jnp.where(new_hit[:, None], nrm, normal)
        hit = hit | new_hit
    out = jnp.concatenate([intersection, normal], axis=-1) * hit[:, None].astype(jnp.float32)
    return out


def _lerp(u, a, b):
    return a + u * (b - a)


def kernel(origins, directions, sdf_values):
    gidx, q, acts = _trajectory(origins, directions)
    gathered = _sc_gather(sdf_values.reshape(_N * _N * _N), gidx)
    return _solve(origins, directions, gathered, q, acts)


# speedup vs baseline: 4.3861x; 1.0199x over previous
import functools

import jax
import jax.numpy as jnp
import numpy as np
from jax import lax
from jax.experimental import pallas as pl
from jax.experimental.pallas import tpu as pltpu
from jax.experimental.pallas import tpu_sc as plsc

_N = 256
_B = 16384
_STEPS = 8

_MIN_B = jnp.array([-1.0, -1.0, -1.0], jnp.float32)
_MAX_B = jnp.array([1.0, 1.0, 1.0], jnp.float32)
_VOXEL = (_MAX_B - _MIN_B) / jnp.array([_N - 1.0] * 3, jnp.float32)
_UPPER = jnp.array([_N - 1] * 3, jnp.int32)
_OFFSETS = jnp.array([[0, 0, 0], [1, 0, 0], [0, 1, 0], [1, 1, 0],
                      [0, 0, 1], [1, 0, 1], [0, 1, 1], [1, 1, 1]], jnp.int32)

_NW = 32
_GPW = 64 * _B // _NW
_ROWS = _GPW // 128


def _trajectory(origins, directions):
    inv_dir = 1.0 / directions
    t1 = (_MIN_B - origins) * inv_dir
    t2 = (_MAX_B - origins) * inv_dir
    t_near = jnp.max(jnp.minimum(t1, t2), axis=-1)
    t_far = jnp.min(jnp.maximum(t1, t2), axis=-1)
    t_near = t_near * (t_near > 0)
    valid = t_near <= t_far
    t_near = jnp.where(valid, t_near, 0.0)
    t_far = jnp.where(valid, t_far, -1.0)
    current_pos = origins + t_near[:, None] * directions
    idx = jnp.floor((current_pos - _MIN_B) / _VOXEL).astype(jnp.int32)
    step = jnp.sign(directions).astype(jnp.int32)
    t_delta = _VOXEL / jnp.abs(directions)
    next_boundary = (idx + jnp.maximum(step, 0)).astype(jnp.float32) * _VOXEL + _MIN_B
    t_next = (next_boundary - origins) / directions
    q_l, act_l = [], []
    for _ in range(_STEPS):
        inside = jnp.all((idx >= 0) & (idx < _UPPER[None, :]), axis=-1)
        qidx = jnp.floor((current_pos - _MIN_B) / _VOXEL).astype(jnp.int32)
        q_l.append(qidx)
        act_l.append(valid & inside)
        t_min = jnp.min(t_next, axis=-1, keepdims=True)
        mask = (t_next < t_far[:, None]) & (t_next == t_min)
        idx = idx + step * mask.astype(jnp.int32)
        current_pos = origins + t_min * directions
        t_next = t_next + t_delta * mask.astype(jnp.float32)
    q = jnp.stack(q_l)
    a = jnp.stack(act_l)
    q, a = lax.optimization_barrier((q, a))
    corners = q[:, None, :, :] + _OFFSETS[None, :, None, :]
    cc = jnp.clip(corners, 0, _UPPER[None, None, None, :])
    flat = cc[..., 0] * (_N * _N) + cc[..., 1] * _N + cc[..., 2]
    gidx = flat.reshape(_NW, _ROWS, 128)
    return gidx, q, a


def _sc_gather_kernel(table_hbm, idx_hbm, out_hbm, idx_v, rows_v, sem):
    wid = lax.axis_index("s") * 2 + lax.axis_index("c")
    pltpu.sync_copy(idx_hbm.at[wid], idx_v)
    nbatch = 8
    rpb = _ROWS // nbatch

    def _fire(j):
        for u in range(rpb):
            r = j * rpb + u
            pltpu.async_copy(table_hbm.at[idx_v.at[r]], rows_v.at[r], sem)

    def _drain(j):
        for u in range(rpb):
            r = j * rpb + u
            pltpu.make_async_copy(table_hbm.at[idx_v.at[r]], rows_v.at[r],
                                  sem).wait()

    _fire(0)

    def _body(j, carry):
        _fire(j)
        _drain(j - 1)
        return carry

    lax.fori_loop(1, nbatch, _body, 0)
    _drain(nbatch - 1)
    pltpu.sync_copy(rows_v, out_hbm.at[wid])


def _sc_gather(flat_sdf, idx):
    gk = functools.partial(
        pl.kernel,
        mesh=plsc.VectorSubcoreMesh(core_axis_name="c", subcore_axis_name="s"),
        out_type=jax.ShapeDtypeStruct((_NW, _ROWS, 128), jnp.float32),
        scratch_types=[
            pltpu.VMEM((_ROWS, 128), jnp.int32),
            pltpu.VMEM((_ROWS, 128), jnp.float32),
            pltpu.SemaphoreType.DMA,
        ],
    )(_sc_gather_kernel)
    return gk(flat_sdf, idx)


def _solve(origins, directions, gathered, q, acts):
    gathered, q, acts = lax.optimization_barrier((gathered, q, acts))
    sdf3 = gathered.reshape(_STEPS, 8, _B)
    hit = jnp.zeros((_B,), bool)
    intersection = jnp.zeros_like(origins)
    normal = jnp.zeros_like(origins)
    for s in range(_STEPS):
        sdf = sdf3[s]
        corners = q[s][:, None, :] + _OFFSETS[None, :, :]
        positions = _MIN_B + corners.astype(jnp.float32) * _VOXEL
        act = acts[s] & (~hit)
        s000, s100, s010, s110, s001, s101, s011, s111 = [sdf[i] for i in range(8)]
        px000 = positions[:, 0, 0]; py000 = positions[:, 0, 1]; pz000 = positions[:, 0, 2]
        px100 = positions[:, 1, 0]; py010 = positions[:, 2, 1]; pz001 = positions[:, 4, 2]
        ox = (origins[:, 0] - px000) / (px100 - px000)
        oy = (origins[:, 1] - py000) / (py010 - py000)
        oz = (origins[:, 2] - pz000) / (pz001 - pz000)
        dx = directions[:, 0] / (px100 - px000)
        dy = directions[:, 1] / (py010 - py000)
        dz = directions[:, 2] / (pz001 - pz000)
        k0 = s000; k1 = s100 - s000; k2 = s010 - s000; k3 = s110 - s010 - k1
        k4 = k0 - s001
        a = s101 - s001
        k5 = k1 - a
        k6 = k2 - (s011 - s001)
        k7 = k3 - (s111 - s011 - a)
        m0 = ox * oy; m1 = dx * dy; m2 = ox * dy + oy * dx
        m3 = k5 * oz - k1; m4 = k6 * oz - k2; m5 = k7 * oz - k3
        c0 = k4 * oz - k0 + ox * m3 + oy * m4 + m0 * m5
        c1 = dx * m3 + dy * m4 + m2 * m5 + dz * (k4 + k5 * ox + k6 * oy + k7 * m0)
        c2 = m1 * m5 + dz * (k5 * dx + k6 * dy + k7 * m2)
        c3 = k7 * m1 * dz

        lin_valid = c1 > 1e-19
        c1s = jnp.where(lin_valid, c1, 1.0)
        t_lin = -c0 / c1s
        use_lin = c2 <= 1e-4
        c2s = jnp.where(use_lin, 1.0, c2)
        disc = c1 ** 2 - 4.0 * c2 * c0
        sq = jnp.sqrt(jnp.maximum(disc, 0.0))
        r1 = (-c1 + sq) / (2.0 * c2s)
        r2 = (-c1 - sq) / (2.0 * c2s)
        rmin = jnp.minimum(r1, r2)
        t_q = jnp.where(use_lin, t_lin, rmin)
        v_q = jnp.where(use_lin, lin_valid, disc > 0)

        use_quad = c3 <= 1e-4
        c3s = jnp.where(use_quad, 1.0, c3)
        aa = c2 / c3s
        bb = c1 / c3s
        cc_ = c0 / c3s
        Q = (aa ** 2 - 3.0 * bb) / 9.0
        Rr = (2.0 * aa ** 3 - 9.0 * aa * bb + 27.0 * cc_) / 54.0
        trig = Rr ** 2 < Q ** 3
        Qp = jnp.maximum(Q, 1e-12)
        ratio = jnp.clip(Rr / jnp.sqrt(Qp ** 3), -0.999999, 0.999999)
        theta = jnp.arccos(ratio)
        sqrt_Q = jnp.sqrt(Qp)
        t_trig = -2.0 * sqrt_Q * jnp.cos(theta / 3.0) - aa / 3.0
        arg = jnp.maximum(Rr ** 2 - Q ** 3, 0.0)
        base = jnp.maximum(jnp.abs(Rr) + jnp.sqrt(arg), 1e-12)
        A = -jnp.sign(Rr) * base ** (1.0 / 3.0)
        nonzero_A = jnp.abs(A) > 0
        As = jnp.where(nonzero_A, A, 1.0)
        Bv = jnp.where(nonzero_A, Q / As, 0.0)
        t_card = A + Bv - aa / 3.0
        t_cub = jnp.where(trig, t_trig, t_card)
        t = jnp.where(use_quad, t_q, t_cub)
        root_valid = jnp.where(use_quad, v_q, jnp.ones_like(v_q))

        new_hit = act & root_valid
        inter = origins + t[:, None] * directions

        x = (inter[:, 0] - px000) / (px100 - px000)
        y = (inter[:, 1] - py000) / (py010 - py000)
        z = (inter[:, 2] - pz000) / (pz001 - pz000)
        y0 = _lerp(y, sdf[1] - sdf[0], sdf[3] - sdf[2])
        y1 = _lerp(y, sdf[5] - sdf[4], sdf[7] - sdf[6])
        df_dx = _lerp(z, y0, y1)
        x0 = _lerp(x, sdf[2] - sdf[0], sdf[3] - sdf[1])
        x1 = _lerp(x, sdf[6] - sdf[4], sdf[7] - sdf[5])
        df_dy = _lerp(z, x0, x1)
        x0 = _lerp(x, sdf[4] - sdf[0], sdf[5] - sdf[1])
        x1 = _lerp(x, sdf[6] - sdf[2], sdf[7] - sdf[3])
        df_dz = _lerp(y, x0, x1)
        nrm = jnp.stack([df_dx, df_dy, df_dz], axis=-1)

        intersection = jnp.where(new_hit[:, None], inter, intersection)
        normal = jnp.where(new_hit[:, None], nrm, normal)
        hit = hit | new_hit
    out = jnp.concatenate([intersection, normal], axis=-1) * hit[:, None].astype(jnp.float32)
    return out


def _lerp(u, a, b):
    return a + u * (b - a)


def kernel(origins, directions, sdf_values):
    gidx, q, acts = _trajectory(origins, directions)
    gathered = _sc_gather(sdf_values.reshape(_N * _N * _N), gidx)
    return _solve(origins, directions, gathered, q, acts)
